# baseline (speedup 1.0000x reference)
import functools

import jax
import jax.numpy as jnp
from jax import lax
from jax.experimental import pallas as pl
from jax.experimental.pallas import tpu as pltpu
from jax.experimental.pallas import tpu_sc as plsc

NUM_HIDDEN = 32
BATCH = 16384
NC = 2
NS = 16
NW = NC * NS
B_PER_W = BATCH // NW
CH = 128
NCH = B_PER_W // CH
L = 16
RPS = 4
SROWS = 1000000 // RPS
SWIDE = RPS * NUM_HIDDEN


def _make_sc_kernel():
    mesh = plsc.VectorSubcoreMesh(core_axis_name="c", subcore_axis_name="s")

    @functools.partial(
        pl.kernel,
        mesh=mesh,
        compiler_params=pltpu.CompilerParams(needs_layout_passes=False),
        out_type=jax.ShapeDtypeStruct((BATCH,), jnp.float32),
        scratch_types=[
            pltpu.VMEM((B_PER_W,), jnp.int32),
            pltpu.VMEM((B_PER_W,), jnp.int32),
            pltpu.VMEM((CH,), jnp.int32),
            pltpu.VMEM((CH,), jnp.int32),
            pltpu.VMEM((CH, SWIDE), jnp.float32),
            pltpu.VMEM((CH, SWIDE), jnp.float32),
            pltpu.VMEM((B_PER_W,), jnp.float32),
            pltpu.SemaphoreType.DMA,
            pltpu.SemaphoreType.DMA,
        ],
    )
    def sc_kernel(uidx_hbm, iidx_hbm, user_hbm, item_hbm, out_hbm,
                  uidx_v, iidx_v, utidx_v, itidx_v, urows_v, irows_v, out_v,
                  sem_u, sem_i):
        wid = lax.axis_index("s") * NC + lax.axis_index("c")
        base = wid * B_PER_W
        row_iota = lax.iota(jnp.int32, L)

        pltpu.sync_copy(uidx_hbm.at[pl.ds(base, B_PER_W)], uidx_v)
        pltpu.sync_copy(iidx_hbm.at[pl.ds(base, B_PER_W)], iidx_v)

        def chunk_body(c, carry):
            cb = c * CH

            def tidx_body(g, cc):
                uvec = uidx_v[pl.ds(cb + g * L, L)]
                ivec = iidx_v[pl.ds(cb + g * L, L)]
                utidx_v[pl.ds(g * L, L)] = uvec >> 2
                itidx_v[pl.ds(g * L, L)] = ivec >> 2
                return cc

            lax.fori_loop(0, CH // L, tidx_body, 0)

            cp_u = pltpu.async_copy(user_hbm.at[utidx_v], urows_v, sem_u)
            cp_i = pltpu.async_copy(item_hbm.at[itidx_v], irows_v, sem_i)
            cp_u.wait()
            cp_i.wait()

            def group_body(g, cc):
                pos = g * L + row_iota
                uvec = uidx_v[pl.ds(cb + g * L, L)]
                ivec = iidx_v[pl.ds(cb + g * L, L)]
                ju = (uvec & 3) * NUM_HIDDEN
                ji = (ivec & 3) * NUM_HIDDEN
                acc = jnp.zeros((L,), jnp.float32)
                for h in range(NUM_HIDDEN):
                    u = plsc.load_gather(urows_v, [pos, ju + h])
                    v = plsc.load_gather(irows_v, [pos, ji + h])
                    acc = acc + u * v
                out_v[pl.ds(cb + g * L, L)] = acc
                return cc

            lax.fori_loop(0, CH // L, group_body, 0)
            return carry

        lax.fori_loop(0, NCH, chunk_body, 0)
        pltpu.sync_copy(out_v, out_hbm.at[pl.ds(base, B_PER_W)])

    return sc_kernel


_SC_KERNEL = _make_sc_kernel()


@jax.jit
def kernel(indices, ratings, user_table, item_table):
    idx = indices.astype(jnp.int32)
    u2 = user_table.reshape(SROWS, SWIDE)
    i2 = item_table.reshape(SROWS, SWIDE)
    pred = _SC_KERNEL(idx[0], idx[1], u2, i2)
    return (pred, ratings)
